# Initial kernel scaffold; baseline (speedup 1.0000x reference)
import jax
import jax.numpy as jnp
from jax.experimental import pallas as pl


def kernel(phi_par, X):
    raise NotImplementedError("write your pallas kernel here")


# speedup vs baseline: 196.9301x; 196.9301x over previous
import functools

import jax
import jax.numpy as jnp
from jax import lax
from jax.experimental import pallas as pl
from jax.experimental.pallas import tpu as pltpu
from jax.experimental.pallas import tpu_sc as plsc

_N_ROWS = 100000
_L = 102
_NTRI = _L - 2
_R = 3
_LANES = 16
_NW = 32
_NB = _N_ROWS // _LANES
_XW = _LANES * _L
_GW = _LANES * _R


def _tc_table_body(p_ref, phi_ref, u_ref):
    x = p_ref[...]
    m = jnp.max(x, axis=1, keepdims=True)
    e = jnp.exp(x - m)
    s = jnp.sum(e, axis=1, keepdims=True)
    phi_ref[...] = e / s
    t = (x - m) - jnp.log(s)
    u_ref[0:16, :] = t[16:32, :] - t[0:16, :]
    u_ref[16:32, :] = t[32:48, :] - t[0:16, :]


def _sc_gamma_body(x_hbm, u_hbm, out_hbm, u_v, x_v, g_v):
    wid = lax.axis_index("s") * 2 + lax.axis_index("c")
    pltpu.sync_copy(u_hbm, u_v)
    lane = lax.iota(jnp.int32, _LANES)
    lane_off = lane * _L
    out_base = lane * _R
    nblk = (_NB + _NW - 1 - wid) // _NW

    def block_body(k, carry):
        blk = wid + k * _NW
        pltpu.sync_copy(x_hbm.at[pl.ds(blk * _XW, _XW)], x_v)
        x0 = plsc.load_gather(x_v, [lane_off])
        x1 = plsc.load_gather(x_v, [lane_off + 1])
        t = x0 * 4 + x1
        acc1 = jnp.zeros((_LANES,), jnp.float32)
        acc2 = jnp.zeros((_LANES,), jnp.float32)
        for l in range(2, _L):
            xl = plsc.load_gather(x_v, [lane_off + l])
            t = (t & 15) * 4 + xl
            acc1 = acc1 + plsc.load_gather(u_v, [t])
            acc2 = acc2 + plsc.load_gather(u_v, [t + 64])
        d1 = acc1 * (1.0 / _NTRI)
        d2 = acc2 * (1.0 / _NTRI)
        m = jnp.maximum(jnp.maximum(d1, d2), 0.0)
        e0 = jnp.exp(-m)
        e1 = jnp.exp(d1 - m)
        e2 = jnp.exp(d2 - m)
        inv = 1.0 / (e0 + e1 + e2)
        plsc.store_scatter(g_v, [out_base], e0 * inv)
        plsc.store_scatter(g_v, [out_base + 1], e1 * inv)
        plsc.store_scatter(g_v, [out_base + 2], e2 * inv)
        pltpu.sync_copy(g_v, out_hbm.at[pl.ds(blk * _GW, _GW)])
        return carry

    lax.fori_loop(0, nblk, block_body, 0)


def kernel(phi_par, X):
    p48 = phi_par.astype(jnp.float32).reshape(48, 4)
    p_pad = jnp.pad(p48, ((0, 0), (0, 124)), constant_values=-1e30)
    phi_pad, u_pad = pl.pallas_call(
        _tc_table_body,
        out_shape=[
            jax.ShapeDtypeStruct((48, 128), jnp.float32),
            jax.ShapeDtypeStruct((32, 128), jnp.float32),
        ],
    )(p_pad)
    phi = phi_pad[:, :4].reshape(3, 4, 4, 4)
    u_flat = u_pad[:, :4].reshape(128)

    mesh = plsc.VectorSubcoreMesh(core_axis_name="c", subcore_axis_name="s")
    sc = functools.partial(
        pl.kernel,
        mesh=mesh,
        out_type=jax.ShapeDtypeStruct((_N_ROWS * _R,), jnp.float32),
        scratch_types=[
            pltpu.VMEM((128,), jnp.float32),
            pltpu.VMEM((_XW,), jnp.int32),
            pltpu.VMEM((_GW,), jnp.float32),
        ],
        compiler_params=pltpu.CompilerParams(needs_layout_passes=False),
    )(_sc_gamma_body)
    gamma = sc(X.reshape(-1), u_flat).reshape(_N_ROWS, _R)
    return phi, gamma


# speedup vs baseline: 241.6109x; 1.2269x over previous
import functools

import jax
import jax.numpy as jnp
from jax import lax
from jax.experimental import pallas as pl
from jax.experimental.pallas import tpu as pltpu
from jax.experimental.pallas import tpu_sc as plsc

_N_ROWS = 100000
_L = 102
_NTRI = _L - 2
_R = 3
_LANES = 16
_NW = 32
_NB = _N_ROWS // _LANES
_XW = _LANES * _L
_GW = _LANES * _R


def _tc_table_body(p_ref, phi_ref, u_ref):
    x = p_ref[...]
    m = jnp.max(x, axis=1, keepdims=True)
    e = jnp.exp(x - m)
    s = jnp.sum(e, axis=1, keepdims=True)
    phi_ref[...] = e / s
    t = (x - m) - jnp.log(s)
    u_ref[0:16, :] = t[16:32, :] - t[0:16, :]
    u_ref[16:32, :] = t[32:48, :] - t[0:16, :]


_NBT = 2 * ((_NB + 2 * _NW - 1) // (2 * _NW))


def _sc_gamma_body(x_hbm, u_hbm, out_hbm,
                   u_v, xa_v, xb_v, ga_v, gb_v, in_a, in_b, o_a, o_b):
    wid = lax.axis_index("s") * 2 + lax.axis_index("c")
    pltpu.sync_copy(u_hbm, u_v)
    lane = lax.iota(jnp.int32, _LANES)
    lane_off = lane * _L
    out_base = lane * _R

    def bidx(k):
        b = wid + k * _NW
        return jnp.where(b >= _NB, b - _NB, b)

    def fetch(k, buf, sem):
        pltpu.async_copy(x_hbm.at[pl.ds(bidx(k) * _XW, _XW)], buf, sem)

    def wait_in(buf, sem):
        pltpu.make_async_copy(x_hbm.at[pl.ds(0, _XW)], buf, sem).wait()

    def compute(xbuf):
        x0 = plsc.load_gather(xbuf, [lane_off])
        x1 = plsc.load_gather(xbuf, [lane_off + 1])
        t = x0 * 4 + x1
        acc1 = jnp.zeros((_LANES,), jnp.float32)
        acc2 = jnp.zeros((_LANES,), jnp.float32)
        for l in range(2, _L):
            xl = plsc.load_gather(xbuf, [lane_off + l])
            t = (t & 15) * 4 + xl
            acc1 = acc1 + plsc.load_gather(u_v, [t])
            acc2 = acc2 + plsc.load_gather(u_v, [t + 64])
        d1 = acc1 * (1.0 / _NTRI)
        d2 = acc2 * (1.0 / _NTRI)
        m = jnp.maximum(jnp.maximum(d1, d2), 0.0)
        e0 = jnp.exp(-m)
        e1 = jnp.exp(d1 - m)
        e2 = jnp.exp(d2 - m)
        inv = 1.0 / (e0 + e1 + e2)
        return e0 * inv, e1 * inv, e2 * inv

    def emit(k, q, gbuf, gsem, vals):
        @pl.when(q > 0)
        def _():
            pltpu.make_async_copy(gbuf, out_hbm.at[pl.ds(0, _GW)], gsem).wait()
        g0, g1, g2 = vals
        plsc.store_scatter(gbuf, [out_base], g0)
        plsc.store_scatter(gbuf, [out_base + 1], g1)
        plsc.store_scatter(gbuf, [out_base + 2], g2)
        pltpu.async_copy(gbuf, out_hbm.at[pl.ds(bidx(k) * _GW, _GW)], gsem)

    fetch(0, xa_v, in_a)

    def pair(q, carry):
        k0 = 2 * q
        wait_in(xa_v, in_a)
        fetch(k0 + 1, xb_v, in_b)
        va = compute(xa_v)
        fetch(k0 + 2, xa_v, in_a)
        emit(k0, q, ga_v, o_a, va)
        wait_in(xb_v, in_b)
        vb = compute(xb_v)
        emit(k0 + 1, q, gb_v, o_b, vb)
        return carry

    lax.fori_loop(0, _NBT // 2, pair, 0)
    wait_in(xa_v, in_a)
    pltpu.make_async_copy(ga_v, out_hbm.at[pl.ds(0, _GW)], o_a).wait()
    pltpu.make_async_copy(gb_v, out_hbm.at[pl.ds(0, _GW)], o_b).wait()


def kernel(phi_par, X):
    p48 = phi_par.astype(jnp.float32).reshape(48, 4)
    p_pad = jnp.pad(p48, ((0, 0), (0, 124)), constant_values=-1e30)
    phi_pad, u_pad = pl.pallas_call(
        _tc_table_body,
        out_shape=[
            jax.ShapeDtypeStruct((48, 128), jnp.float32),
            jax.ShapeDtypeStruct((32, 128), jnp.float32),
        ],
    )(p_pad)
    phi = phi_pad[:, :4].reshape(3, 4, 4, 4)
    u_flat = u_pad[:, :4].reshape(128)

    mesh = plsc.VectorSubcoreMesh(core_axis_name="c", subcore_axis_name="s")
    sc = functools.partial(
        pl.kernel,
        mesh=mesh,
        out_type=jax.ShapeDtypeStruct((_N_ROWS * _R,), jnp.float32),
        scratch_types=[
            pltpu.VMEM((128,), jnp.float32),
            pltpu.VMEM((_XW,), jnp.int32),
            pltpu.VMEM((_XW,), jnp.int32),
            pltpu.VMEM((_GW,), jnp.float32),
            pltpu.VMEM((_GW,), jnp.float32),
            pltpu.SemaphoreType.DMA,
            pltpu.SemaphoreType.DMA,
            pltpu.SemaphoreType.DMA,
            pltpu.SemaphoreType.DMA,
        ],
        compiler_params=pltpu.CompilerParams(needs_layout_passes=False),
    )(_sc_gamma_body)
    gamma = sc(X.reshape(-1), u_flat).reshape(_N_ROWS, _R)
    return phi, gamma


# speedup vs baseline: 319.9429x; 1.3242x over previous
import functools

import jax
import jax.numpy as jnp
from jax import lax
from jax.experimental import pallas as pl
from jax.experimental.pallas import tpu as pltpu
from jax.experimental.pallas import tpu_sc as plsc

_N_ROWS = 100000
_L = 102
_NTRI = _L - 2
_R = 3
_LANES = 16
_NW = 32
_NB = _N_ROWS // _LANES
_NBT = 2 * ((_NB + 2 * _NW - 1) // (2 * _NW))


def _tc_table_body(p_ref, phi_ref, u_ref):
    x = p_ref[...]
    m = jnp.max(x, axis=1, keepdims=True)
    e = jnp.exp(x - m)
    s = jnp.sum(e, axis=1, keepdims=True)
    phi_ref[...] = e / s
    t = (x - m) - jnp.log(s)
    u_ref[0:16, :] = t[16:32, :] - t[0:16, :]
    u_ref[16:32, :] = t[32:48, :] - t[0:16, :]


def _sc_gamma_body(x_hbm, u_hbm, out_hbm,
                   u_v, xa_v, xb_v, ga_v, gb_v, in_a, in_b, o_a, o_b):
    wid = lax.axis_index("s") * 2 + lax.axis_index("c")
    pltpu.sync_copy(u_hbm, u_v)
    lane = lax.iota(jnp.int32, _LANES)

    def bidx(k):
        b = wid + k * _NW
        return jnp.where(b >= _NB, b - _NB, b)

    def fetch(k, buf, sem):
        pltpu.async_copy(x_hbm.at[pl.ds(bidx(k) * _LANES, _LANES)], buf, sem)

    def wait_in(buf, sem):
        pltpu.make_async_copy(x_hbm.at[pl.ds(0, _LANES)], buf, sem).wait()

    def col(l):
        return jnp.full((_LANES,), l, jnp.int32)

    def compute(xbuf):
        x0 = plsc.load_gather(xbuf, [lane, col(0)])
        x1 = plsc.load_gather(xbuf, [lane, col(1)])
        t = x0 * 4 + x1
        acc1 = jnp.zeros((_LANES,), jnp.float32)
        acc2 = jnp.zeros((_LANES,), jnp.float32)
        for l in range(2, _L):
            xl = plsc.load_gather(xbuf, [lane, col(l)])
            t = (t & 15) * 4 + xl
            row = jax.lax.shift_right_logical(t, 2)
            c = t & 3
            acc1 = acc1 + plsc.load_gather(u_v, [row, c])
            acc2 = acc2 + plsc.load_gather(u_v, [row + 16, c])
        d1 = acc1 * (1.0 / _NTRI)
        d2 = acc2 * (1.0 / _NTRI)
        m = jnp.maximum(jnp.maximum(d1, d2), 0.0)
        e0 = jnp.exp(-m)
        e1 = jnp.exp(d1 - m)
        e2 = jnp.exp(d2 - m)
        inv = 1.0 / (e0 + e1 + e2)
        return e0 * inv, e1 * inv, e2 * inv

    def emit(k, q, gbuf, gsem, vals):
        @pl.when(q > 0)
        def _():
            pltpu.make_async_copy(
                gbuf, out_hbm.at[pl.ds(0, _LANES)], gsem).wait()
        g0, g1, g2 = vals
        plsc.store_scatter(gbuf, [lane, col(0)], g0)
        plsc.store_scatter(gbuf, [lane, col(1)], g1)
        plsc.store_scatter(gbuf, [lane, col(2)], g2)
        pltpu.async_copy(
            gbuf, out_hbm.at[pl.ds(bidx(k) * _LANES, _LANES)], gsem)

    fetch(0, xa_v, in_a)

    def pair(q, carry):
        k0 = 2 * q
        wait_in(xa_v, in_a)
        fetch(k0 + 1, xb_v, in_b)
        va = compute(xa_v)
        fetch(k0 + 2, xa_v, in_a)
        emit(k0, q, ga_v, o_a, va)
        wait_in(xb_v, in_b)
        vb = compute(xb_v)
        emit(k0 + 1, q, gb_v, o_b, vb)
        return carry

    lax.fori_loop(0, _NBT // 2, pair, 0)
    wait_in(xa_v, in_a)
    pltpu.make_async_copy(ga_v, out_hbm.at[pl.ds(0, _LANES)], o_a).wait()
    pltpu.make_async_copy(gb_v, out_hbm.at[pl.ds(0, _LANES)], o_b).wait()


def kernel(phi_par, X):
    p48 = phi_par.astype(jnp.float32).reshape(48, 4)
    p_pad = jnp.pad(p48, ((0, 0), (0, 124)), constant_values=-1e30)
    phi_pad, u_tab = pl.pallas_call(
        _tc_table_body,
        out_shape=[
            jax.ShapeDtypeStruct((48, 128), jnp.float32),
            jax.ShapeDtypeStruct((32, 128), jnp.float32),
        ],
    )(p_pad)
    phi = phi_pad[:, :4].reshape(3, 4, 4, 4)

    mesh = plsc.VectorSubcoreMesh(core_axis_name="c", subcore_axis_name="s")
    sc = functools.partial(
        pl.kernel,
        mesh=mesh,
        out_type=jax.ShapeDtypeStruct((_N_ROWS, _R), jnp.float32),
        scratch_types=[
            pltpu.VMEM((32, 128), jnp.float32),
            pltpu.VMEM((_LANES, _L), jnp.int32),
            pltpu.VMEM((_LANES, _L), jnp.int32),
            pltpu.VMEM((_LANES, _R), jnp.float32),
            pltpu.VMEM((_LANES, _R), jnp.float32),
            pltpu.SemaphoreType.DMA,
            pltpu.SemaphoreType.DMA,
            pltpu.SemaphoreType.DMA,
            pltpu.SemaphoreType.DMA,
        ],
        compiler_params=pltpu.CompilerParams(needs_layout_passes=False),
    )(_sc_gamma_body)
    gamma = sc(X, u_tab)
    return phi, gamma


# speedup vs baseline: 394.9537x; 1.2345x over previous
import functools

import jax
import jax.numpy as jnp
from jax import lax
from jax.experimental import pallas as pl
from jax.experimental.pallas import tpu as pltpu
from jax.experimental.pallas import tpu_sc as plsc

_N_ROWS = 100000
_L = 102
_NTRI = _L - 2
_R = 3
_LANES = 16
_NW = 32
_NB = _N_ROWS // _LANES
_NBT = 2 * ((_NB + 2 * _NW - 1) // (2 * _NW))


def _tc_table_body(p_ref, phi_ref, u_ref):
    x = p_ref[...]
    m = jnp.max(x, axis=1, keepdims=True)
    e = jnp.exp(x - m)
    s = jnp.sum(e, axis=1, keepdims=True)
    phi_ref[...] = e / s
    t = (x - m) - jnp.log(s)
    u_ref[0:16, :] = t[16:32, :] - t[0:16, :]
    u_ref[16:32, :] = t[32:48, :] - t[0:16, :]


def _sc_gamma_body(x_hbm, u_hbm, out_hbm,
                   u_v, up_v, xa_v, xb_v, ga_v, gb_v, in_a, in_b, o_a, o_b):
    wid = lax.axis_index("s") * 2 + lax.axis_index("c")
    pltpu.sync_copy(u_hbm, u_v)
    lane = lax.iota(jnp.int32, _LANES)

    for g in range(4):
        tg = lane + g * _LANES
        row = jax.lax.shift_right_logical(tg, 2)
        c = tg & 3
        u1g = plsc.load_gather(u_v, [row, c])
        u2g = plsc.load_gather(u_v, [row + 16, c])
        w = plsc.bitcast(
            plsc.pack(u1g, u2g, format=plsc.PackFormat.INTERLEAVED), jnp.int32)
        up_v[pl.ds(g * _LANES, _LANES)] = w

    def bidx(k):
        b = wid + k * _NW
        return jnp.where(b >= _NB, b - _NB, b)

    def fetch(k, buf, sem):
        pltpu.async_copy(x_hbm.at[pl.ds(bidx(k) * _LANES, _LANES)], buf, sem)

    def wait_in(buf, sem):
        pltpu.make_async_copy(x_hbm.at[pl.ds(0, _LANES)], buf, sem).wait()

    def col(l):
        return jnp.full((_LANES,), l, jnp.int32)

    def compute(xbuf):
        x0 = plsc.load_gather(xbuf, [lane, col(0)])
        x1 = plsc.load_gather(xbuf, [lane, col(1)])
        q = x0 * 4 + x1
        acc1 = jnp.zeros((_LANES,), jnp.float32)
        acc2 = jnp.zeros((_LANES,), jnp.float32)
        xm1 = x1
        for l in range(2, _L):
            xl = plsc.load_gather(xbuf, [lane, col(l)])
            t = q * 4 + xl
            q = xm1 * 4 + xl
            xm1 = xl
            w = plsc.load_gather(up_v, [t])
            acc1 = acc1 + plsc.bitcast(jax.lax.shift_left(w, 16), jnp.float32)
            acc2 = acc2 + plsc.bitcast(w & jnp.int32(-65536), jnp.float32)
        d1 = acc1 * (1.0 / _NTRI)
        d2 = acc2 * (1.0 / _NTRI)
        m = jnp.maximum(jnp.maximum(d1, d2), 0.0)
        e0 = jnp.exp(-m)
        e1 = jnp.exp(d1 - m)
        e2 = jnp.exp(d2 - m)
        inv = 1.0 / (e0 + e1 + e2)
        return e0 * inv, e1 * inv, e2 * inv

    def emit(k, q, gbuf, gsem, vals):
        @pl.when(q > 0)
        def _():
            pltpu.make_async_copy(
                gbuf, out_hbm.at[pl.ds(0, _LANES)], gsem).wait()
        g0, g1, g2 = vals
        plsc.store_scatter(gbuf, [lane, col(0)], g0)
        plsc.store_scatter(gbuf, [lane, col(1)], g1)
        plsc.store_scatter(gbuf, [lane, col(2)], g2)
        pltpu.async_copy(
            gbuf, out_hbm.at[pl.ds(bidx(k) * _LANES, _LANES)], gsem)

    fetch(0, xa_v, in_a)

    def pair(q, carry):
        k0 = 2 * q
        wait_in(xa_v, in_a)
        fetch(k0 + 1, xb_v, in_b)
        va = compute(xa_v)
        fetch(k0 + 2, xa_v, in_a)
        emit(k0, q, ga_v, o_a, va)
        wait_in(xb_v, in_b)
        vb = compute(xb_v)
        emit(k0 + 1, q, gb_v, o_b, vb)
        return carry

    lax.fori_loop(0, _NBT // 2, pair, 0)
    wait_in(xa_v, in_a)
    pltpu.make_async_copy(ga_v, out_hbm.at[pl.ds(0, _LANES)], o_a).wait()
    pltpu.make_async_copy(gb_v, out_hbm.at[pl.ds(0, _LANES)], o_b).wait()


def kernel(phi_par, X):
    p48 = phi_par.astype(jnp.float32).reshape(48, 4)
    p_pad = jnp.pad(p48, ((0, 0), (0, 124)), constant_values=-1e30)
    phi_pad, u_tab = pl.pallas_call(
        _tc_table_body,
        out_shape=[
            jax.ShapeDtypeStruct((48, 128), jnp.float32),
            jax.ShapeDtypeStruct((32, 128), jnp.float32),
        ],
    )(p_pad)
    phi = phi_pad[:, :4].reshape(3, 4, 4, 4)

    mesh = plsc.VectorSubcoreMesh(core_axis_name="c", subcore_axis_name="s")
    sc = functools.partial(
        pl.kernel,
        mesh=mesh,
        out_type=jax.ShapeDtypeStruct((_N_ROWS, _R), jnp.float32),
        scratch_types=[
            pltpu.VMEM((32, 128), jnp.float32),
            pltpu.VMEM((64,), jnp.int32),
            pltpu.VMEM((_LANES, _L), jnp.int32),
            pltpu.VMEM((_LANES, _L), jnp.int32),
            pltpu.VMEM((_LANES, _R), jnp.float32),
            pltpu.VMEM((_LANES, _R), jnp.float32),
            pltpu.SemaphoreType.DMA,
            pltpu.SemaphoreType.DMA,
            pltpu.SemaphoreType.DMA,
            pltpu.SemaphoreType.DMA,
        ],
        compiler_params=pltpu.CompilerParams(needs_layout_passes=False),
    )(_sc_gamma_body)
    gamma = sc(X, u_tab)
    return phi, gamma
